# baseline (speedup 1.0000x reference)
import functools

import jax
import jax.numpy as jnp
from jax import lax
from jax.experimental import pallas as pl
from jax.experimental.pallas import tpu as pltpu
from jax.experimental.pallas import tpu_sc as plsc

B, T, C = 4, 8192, 1024
KTOP = 8
L = 16
NSG = 8
CHUNK = 400
NCHUNK = 20
TAIL = T - NCHUNK * CHUNK
LEAF = 8
NCAND = 10
NRCAND = 10
NEG_INF = float("-inf")
MINKEY = -0x80000000

_SORT8 = [(0, 1), (2, 3), (4, 5), (6, 7),
          (0, 2), (1, 3), (4, 6), (5, 7),
          (1, 2), (5, 6),
          (0, 4), (1, 5), (2, 6), (3, 7),
          (2, 4), (3, 5),
          (1, 2), (3, 4), (5, 6)]


def _orderkey(v):
  b = lax.bitcast_convert_type(v, jnp.int32)
  return b ^ (lax.shift_right_arithmetic(b, 31) & jnp.int32(0x7FFFFFFF))


def _insert_kv(ts, ix, v, iv):
  m = [v > t for t in ts]
  nts, nix = list(ts), list(ix)
  for j in range(KTOP - 1, 0, -1):
    nts[j] = jnp.where(m[j], jnp.where(m[j - 1], ts[j - 1], v), ts[j])
    nix[j] = jnp.where(m[j], jnp.where(m[j - 1], ix[j - 1], iv), ix[j])
  nts[0] = jnp.where(m[0], v, ts[0])
  nix[0] = jnp.where(m[0], iv, ix[0])
  return nts, nix


def _kmax_body(x_hbm, out_hbm, buf0, buf1, csv, csi, obuf, sem0, sem1):
  cid = lax.axis_index("c")
  sid = lax.axis_index("s")
  wid = sid * 2 + cid
  b = wid // NSG
  c0 = (wid % NSG) * 128
  lane = lax.iota(jnp.int32, L)

  def src_for(stage, rows):
    return x_hbm.at[b, pl.ds(stage * CHUNK, rows), pl.ds(c0, 128)]

  def init_sg(sg, carry):
    for j in range(KTOP):
      csv[sg, j] = jnp.full((L,), NEG_INF, jnp.float32)
      csi[sg, j] = jnp.zeros((L,), jnp.int32)
    return carry
  lax.fori_loop(0, NSG, init_sg, 0)

  def process(buf, coff, nleaf, nrows):
    del nrows
    coff_v = jnp.full((L,), coff, jnp.int32)

    def sg_body(sg, carry):
      ts = [csv[sg, j] for j in range(KTOP)]
      ix = [csi[sg, j] for j in range(KTOP)]
      csl = sg * L

      def leaf_body(lb, bst):
        base = lb * LEAF
        bm = buf[base, pl.ds(csl, L)]
        for r in range(1, LEAF):
          bm = jnp.maximum(bm, buf[base + r, pl.ds(csl, L)])
        key = (_orderkey(bm) & jnp.int32(~0x7F)) | (jnp.int32(127) - lb)
        nb = list(bst)
        nb[0] = jnp.maximum(bst[0], key)
        for j in range(1, NCAND):
          nb[j] = jnp.maximum(bst[j], jnp.minimum(key, bst[j - 1]))
        return tuple(nb)

      binit = tuple(jnp.full((L,), MINKEY, jnp.int32) for _ in range(NCAND))
      bst = lax.fori_loop(0, nleaf, leaf_body, binit)
      brow = [(jnp.int32(127) - (k & jnp.int32(0x7F))) * LEAF for k in bst]
      auxb = [jnp.int32(511) - r for r in brow]

      def cand_body(r, rst):
        rv = jnp.full((L,), r, jnp.int32)
        for j in range(NCAND):
          v = plsc.load_gather(buf, [brow[j] + rv, lane + csl])
          key = (_orderkey(v) & jnp.int32(~0x1FF)) | (auxb[j] - rv)
          nr = list(rst)
          nr[0] = jnp.maximum(rst[0], key)
          for s in range(1, NRCAND):
            nr[s] = jnp.maximum(rst[s], jnp.minimum(key, rst[s - 1]))
          rst = tuple(nr)
        return rst

      rinit = tuple(jnp.full((L,), MINKEY, jnp.int32) for _ in range(NRCAND))
      rst = lax.fori_loop(0, LEAF, cand_body, rinit)

      for j in range(NRCAND):
        lrow = jnp.int32(511) - (rst[j] & jnp.int32(0x1FF))
        v = plsc.load_gather(buf, [lrow, lane + csl])
        ts, ix = _insert_kv(ts, ix, v, lrow + coff_v)
      for j in range(KTOP):
        csv[sg, j] = ts[j]
        csi[sg, j] = ix[j]
      return carry

    lax.fori_loop(0, NSG, sg_body, 0)

  pltpu.async_copy(src_for(0, CHUNK), buf0, sem0)
  pltpu.async_copy(src_for(1, CHUNK), buf1, sem1)

  def main_body(i, carry):
    for (buf, sem, off) in ((buf0, sem0, 0), (buf1, sem1, 1)):
      stage = 2 * i + off
      pltpu.make_async_copy(src_for(stage, CHUNK), buf, sem).wait()
      process(buf, stage * CHUNK, CHUNK // LEAF, CHUNK)
      pltpu.async_copy(src_for(jnp.minimum(stage + 2, NCHUNK - 1), CHUNK),
                       buf, sem)
    return carry

  lax.fori_loop(0, NCHUNK // 2, main_body, 0)
  pltpu.make_async_copy(src_for(NCHUNK - 1, CHUNK), buf0, sem0).wait()
  pltpu.make_async_copy(src_for(NCHUNK - 1, CHUNK), buf1, sem1).wait()

  pltpu.sync_copy(x_hbm.at[b, pl.ds(NCHUNK * CHUNK, TAIL), pl.ds(c0, 128)],
                  buf0.at[pl.ds(0, TAIL)])
  process(buf0, NCHUNK * CHUNK, TAIL // LEAF, TAIL)

  def fin_sg(sg, carry):
    ts = [csv[sg, j] for j in range(KTOP)]
    ix = [csi[sg, j] for j in range(KTOP)]
    for (a, d) in _SORT8:
      swap = ix[a] > ix[d]
      ix[a], ix[d] = (jnp.where(swap, ix[d], ix[a]),
                      jnp.where(swap, ix[a], ix[d]))
      ts[a], ts[d] = (jnp.where(swap, ts[d], ts[a]),
                      jnp.where(swap, ts[a], ts[d]))
    for j in range(KTOP):
      obuf[j, pl.ds(sg * L, L)] = ts[j]
    return carry
  lax.fori_loop(0, NSG, fin_sg, 0)

  pltpu.sync_copy(obuf, out_hbm.at[b, :, pl.ds(c0, 128)])


@functools.partial(jax.jit, static_argnames=("k",))
def _kmax(x, k):
  del k
  f = pl.kernel(
      _kmax_body,
      out_type=jax.ShapeDtypeStruct((B, KTOP, C), jnp.float32),
      mesh=plsc.VectorSubcoreMesh(core_axis_name="c", subcore_axis_name="s"),
      scratch_types=[
          pltpu.VMEM((CHUNK, 128), jnp.float32),
          pltpu.VMEM((CHUNK, 128), jnp.float32),
          pltpu.VMEM((NSG, KTOP, L), jnp.float32),
          pltpu.VMEM((NSG, KTOP, L), jnp.int32),
          pltpu.VMEM((KTOP, 128), jnp.float32),
          pltpu.SemaphoreType.DMA,
          pltpu.SemaphoreType.DMA,
      ],
      compiler_params=pltpu.CompilerParams(needs_layout_passes=False),
  )
  return f(x)


def kernel(x, k):
  return _kmax(x, 8)


# speedup vs baseline: 1.0674x; 1.0674x over previous
import functools

import jax
import jax.numpy as jnp
from jax import lax
from jax.experimental import pallas as pl
from jax.experimental.pallas import tpu as pltpu
from jax.experimental.pallas import tpu_sc as plsc

B, T, C = 4, 8192, 1024
KTOP = 8
L = 16
NSG = 8
CHUNK = 400
NCHUNK = 20
TAIL = T - NCHUNK * CHUNK
LEAF = 4
NCAND = 10
NEG_INF = float("-inf")
MINKEY = -0x80000000

_SORT8 = [(0, 1), (2, 3), (4, 5), (6, 7),
          (0, 2), (1, 3), (4, 6), (5, 7),
          (1, 2), (5, 6),
          (0, 4), (1, 5), (2, 6), (3, 7),
          (2, 4), (3, 5),
          (1, 2), (3, 4), (5, 6)]


def _orderkey(v):
  b = lax.bitcast_convert_type(v, jnp.int32)
  return b ^ (lax.shift_right_arithmetic(b, 31) & jnp.int32(0x7FFFFFFF))


def _insert_kv(ts, ix, v, iv):
  m = [v > t for t in ts]
  nts, nix = list(ts), list(ix)
  for j in range(KTOP - 1, 0, -1):
    nts[j] = jnp.where(m[j], jnp.where(m[j - 1], ts[j - 1], v), ts[j])
    nix[j] = jnp.where(m[j], jnp.where(m[j - 1], ix[j - 1], iv), ix[j])
  nts[0] = jnp.where(m[0], v, ts[0])
  nix[0] = jnp.where(m[0], iv, ix[0])
  return nts, nix


def _kmax_body(x_hbm, out_hbm, buf0, buf1, csv, csi, obuf, sem0, sem1):
  cid = lax.axis_index("c")
  sid = lax.axis_index("s")
  wid = sid * 2 + cid
  b = wid // NSG
  c0 = (wid % NSG) * 128
  lane = lax.iota(jnp.int32, L)

  def src_for(stage, rows):
    return x_hbm.at[b, pl.ds(stage * CHUNK, rows), pl.ds(c0, 128)]

  def init_sg(sg, carry):
    for j in range(KTOP):
      csv[sg, j] = jnp.full((L,), NEG_INF, jnp.float32)
      csi[sg, j] = jnp.zeros((L,), jnp.int32)
    return carry
  lax.fori_loop(0, NSG, init_sg, 0)

  def process(buf, coff, nleaf, nrows):
    del nrows
    coff_v = jnp.full((L,), coff, jnp.int32)

    def sg_body(sg, carry):
      ts = [csv[sg, j] for j in range(KTOP)]
      ix = [csi[sg, j] for j in range(KTOP)]
      csl = sg * L

      def leaf_body(lb, bst):
        base = lb * LEAF
        bm = buf[base, pl.ds(csl, L)]
        for r in range(1, LEAF):
          bm = jnp.maximum(bm, buf[base + r, pl.ds(csl, L)])
        key = (_orderkey(bm) & jnp.int32(~0x7F)) | (jnp.int32(127) - lb)
        nb = list(bst)
        nb[0] = jnp.maximum(bst[0], key)
        for j in range(1, NCAND):
          nb[j] = jnp.maximum(bst[j], jnp.minimum(key, bst[j - 1]))
        return tuple(nb)

      binit = tuple(jnp.full((L,), MINKEY, jnp.int32) for _ in range(NCAND))
      bst = lax.fori_loop(0, nleaf, leaf_body, binit)
      brow = [(jnp.int32(127) - (k & jnp.int32(0x7F))) * LEAF for k in bst]

      def cand_body(r, st):
        ts, ix = list(st[:KTOP]), list(st[KTOP:])
        rv = jnp.full((L,), r, jnp.int32)
        for j in range(NCAND):
          lrow = brow[j] + rv
          v = plsc.load_gather(buf, [lrow, lane + csl])
          ts, ix = _insert_kv(ts, ix, v, lrow + coff_v)
        return tuple(ts) + tuple(ix)

      st = lax.fori_loop(0, LEAF, cand_body, tuple(ts) + tuple(ix))
      for j in range(KTOP):
        csv[sg, j] = st[j]
        csi[sg, j] = st[KTOP + j]
      return carry

    lax.fori_loop(0, NSG, sg_body, 0)

  pltpu.async_copy(src_for(0, CHUNK), buf0, sem0)
  pltpu.async_copy(src_for(1, CHUNK), buf1, sem1)

  def main_body(i, carry):
    for (buf, sem, off) in ((buf0, sem0, 0), (buf1, sem1, 1)):
      stage = 2 * i + off
      pltpu.make_async_copy(src_for(stage, CHUNK), buf, sem).wait()
      process(buf, stage * CHUNK, CHUNK // LEAF, CHUNK)
      pltpu.async_copy(src_for(jnp.minimum(stage + 2, NCHUNK - 1), CHUNK),
                       buf, sem)
    return carry

  lax.fori_loop(0, NCHUNK // 2, main_body, 0)
  pltpu.make_async_copy(src_for(NCHUNK - 1, CHUNK), buf0, sem0).wait()
  pltpu.make_async_copy(src_for(NCHUNK - 1, CHUNK), buf1, sem1).wait()

  pltpu.sync_copy(x_hbm.at[b, pl.ds(NCHUNK * CHUNK, TAIL), pl.ds(c0, 128)],
                  buf0.at[pl.ds(0, TAIL)])
  process(buf0, NCHUNK * CHUNK, TAIL // LEAF, TAIL)

  def fin_sg(sg, carry):
    ts = [csv[sg, j] for j in range(KTOP)]
    ix = [csi[sg, j] for j in range(KTOP)]
    for (a, d) in _SORT8:
      swap = ix[a] > ix[d]
      ix[a], ix[d] = (jnp.where(swap, ix[d], ix[a]),
                      jnp.where(swap, ix[a], ix[d]))
      ts[a], ts[d] = (jnp.where(swap, ts[d], ts[a]),
                      jnp.where(swap, ts[a], ts[d]))
    for j in range(KTOP):
      obuf[j, pl.ds(sg * L, L)] = ts[j]
    return carry
  lax.fori_loop(0, NSG, fin_sg, 0)

  pltpu.sync_copy(obuf, out_hbm.at[b, :, pl.ds(c0, 128)])


@functools.partial(jax.jit, static_argnames=("k",))
def _kmax(x, k):
  del k
  f = pl.kernel(
      _kmax_body,
      out_type=jax.ShapeDtypeStruct((B, KTOP, C), jnp.float32),
      mesh=plsc.VectorSubcoreMesh(core_axis_name="c", subcore_axis_name="s"),
      scratch_types=[
          pltpu.VMEM((CHUNK, 128), jnp.float32),
          pltpu.VMEM((CHUNK, 128), jnp.float32),
          pltpu.VMEM((NSG, KTOP, L), jnp.float32),
          pltpu.VMEM((NSG, KTOP, L), jnp.int32),
          pltpu.VMEM((KTOP, 128), jnp.float32),
          pltpu.SemaphoreType.DMA,
          pltpu.SemaphoreType.DMA,
      ],
      compiler_params=pltpu.CompilerParams(needs_layout_passes=False),
  )
  return f(x)


def kernel(x, k):
  return _kmax(x, 8)


# speedup vs baseline: 1.1728x; 1.0987x over previous
import functools

import jax
import jax.numpy as jnp
from jax import lax
from jax.experimental import pallas as pl
from jax.experimental.pallas import tpu as pltpu
from jax.experimental.pallas import tpu_sc as plsc

B, T, C = 4, 8192, 1024
KTOP = 8
L = 16
NW = 32
NGRP = B * (C // L)
GPW = NGRP // NW
CHUNK = 2048
NCHUNK = T // CHUNK
NSTAGE = GPW * NCHUNK
LEAF = 16
NLEAF = CHUNK // LEAF
NCAND = 10
NEG_INF = float("-inf")
MINKEY = -0x80000000

_SORT8 = [(0, 1), (2, 3), (4, 5), (6, 7),
          (0, 2), (1, 3), (4, 6), (5, 7),
          (1, 2), (5, 6),
          (0, 4), (1, 5), (2, 6), (3, 7),
          (2, 4), (3, 5),
          (1, 2), (3, 4), (5, 6)]


def _insert(state, n, v, iv):
  ts, ix = state[:n], state[n:]
  m = [v > t for t in ts]
  new_ts, new_ix = list(ts), list(ix)
  for j in range(n - 1, 0, -1):
    new_ts[j] = jnp.where(m[j], jnp.where(m[j - 1], ts[j - 1], v), ts[j])
    new_ix[j] = jnp.where(m[j], jnp.where(m[j - 1], ix[j - 1], iv), ix[j])
  new_ts[0] = jnp.where(m[0], v, ts[0])
  new_ix[0] = jnp.where(m[0], iv, ix[0])
  return tuple(new_ts) + tuple(new_ix)


def _kmax_body(x_hbm, out_hbm, buf0, buf1, obuf, sem0, sem1):
  cid = lax.axis_index("c")
  sid = lax.axis_index("s")
  wid = sid * 2 + cid
  lane = lax.iota(jnp.int32, L)

  def src_for(stage):
    grp = wid * GPW + stage // NCHUNK
    b = grp // (C // L)
    c0 = (grp % (C // L)) * L
    coff = (stage % NCHUNK) * CHUNK
    return x_hbm.at[b, pl.ds(coff, CHUNK), pl.ds(c0, L)], b, c0, coff

  def issue(stage, buf, sem):
    src, _, _, _ = src_for(jnp.minimum(stage, NSTAGE - 1))
    pltpu.async_copy(src, buf, sem)

  def sub_stage(stage, buf, sem, carry):
    src, b, c0, coff = src_for(stage)
    pltpu.make_async_copy(src, buf, sem).wait()
    cidx = stage % NCHUNK
    g = stage // NCHUNK
    first = cidx == 0
    st = tuple(jnp.where(first, NEG_INF, t) for t in carry[:KTOP]) + \
         tuple(jnp.where(first, 0, i) for i in carry[KTOP:])

    def leaf_body(lb, bst):
      base = lb * LEAF
      bm = buf[base]
      for r in range(1, LEAF):
        bm = jnp.maximum(bm, buf[base + r])
      kb = lax.bitcast_convert_type(bm, jnp.int32)
      kb = kb ^ (lax.shift_right_arithmetic(kb, 31) & jnp.int32(0x7FFFFFFF))
      key = (kb & jnp.int32(~0x7F)) | (jnp.int32(127) - lb)
      nb = list(bst)
      nb[0] = jnp.maximum(bst[0], key)
      for j in range(1, NCAND):
        nb[j] = jnp.maximum(bst[j], jnp.minimum(key, bst[j - 1]))
      return tuple(nb)

    binit = tuple(jnp.full((L,), MINKEY, jnp.int32) for _ in range(NCAND))
    bst = lax.fori_loop(0, NLEAF, leaf_body, binit)
    brow = [(jnp.int32(127) - (k & jnp.int32(0x7F))) * LEAF for k in bst]
    coff_v = jnp.full((L,), coff, jnp.int32)

    def cand_body(r, st):
      rv = jnp.full((L,), r, jnp.int32)
      for j in range(NCAND):
        lrow = brow[j] + rv
        v = plsc.load_gather(buf, [lrow, lane])
        st = _insert(st, KTOP, v, lrow + coff_v)
      return st

    st = lax.fori_loop(0, LEAF, cand_body, st)

    issue(stage + 2, buf, sem)

    ts, ix = list(st[:KTOP]), list(st[KTOP:])
    for (a, d) in _SORT8:
      swap = ix[a] > ix[d]
      ix[a], ix[d] = (jnp.where(swap, ix[d], ix[a]),
                      jnp.where(swap, ix[a], ix[d]))
      ts[a], ts[d] = (jnp.where(swap, ts[d], ts[a]),
                      jnp.where(swap, ts[a], ts[d]))
    for j in range(KTOP):
      obuf[g * KTOP + j] = ts[j]
    return st

  issue(0, buf0, sem0)
  issue(1, buf1, sem1)

  def main_body(i, carry):
    carry = sub_stage(2 * i, buf0, sem0, carry)
    carry = sub_stage(2 * i + 1, buf1, sem1, carry)
    return carry

  init = tuple(jnp.full((L,), NEG_INF, jnp.float32) for _ in range(KTOP)) \
      + tuple(jnp.zeros((L,), jnp.int32) for _ in range(KTOP))
  lax.fori_loop(0, NSTAGE // 2, main_body, init)

  tail, _, _, _ = src_for(NSTAGE - 1)
  pltpu.make_async_copy(tail, buf0, sem0).wait()
  pltpu.make_async_copy(tail, buf1, sem1).wait()

  for g in range(GPW):
    grp = wid * GPW + g
    b = grp // (C // L)
    c0 = (grp % (C // L)) * L
    pltpu.sync_copy(obuf.at[pl.ds(g * KTOP, KTOP)],
                    out_hbm.at[b, :, pl.ds(c0, L)])


@functools.partial(jax.jit, static_argnames=("k",))
def _kmax(x, k):
  del k
  f = pl.kernel(
      _kmax_body,
      out_type=jax.ShapeDtypeStruct((B, KTOP, C), jnp.float32),
      mesh=plsc.VectorSubcoreMesh(core_axis_name="c", subcore_axis_name="s"),
      scratch_types=[
          pltpu.VMEM((CHUNK, L), jnp.float32),
          pltpu.VMEM((CHUNK, L), jnp.float32),
          pltpu.VMEM((GPW * KTOP, L), jnp.float32),
          pltpu.SemaphoreType.DMA,
          pltpu.SemaphoreType.DMA,
      ],
      compiler_params=pltpu.CompilerParams(use_tc_tiling_on_sc=False,
                                           needs_layout_passes=False),
  )
  return f(x)


def kernel(x, k):
  return _kmax(x, 8)
